# baseline (speedup 1.0000x reference)
import jax
import jax.numpy as jnp
from jax import lax
from jax.experimental import pallas as pl
from jax.experimental.pallas import tpu as pltpu
from jax.experimental.pallas import tpu_sc as plsc

IMG = 8192
N = 24576
NC = 2
NS = 16
L = 16
NW = NC * NS
PER = N // NW
CH = 128
NCH = PER // CH
NV = PER // L


def _body(img_hbm, r_hbm, c_hbm, len_hbm, out_hbm, parts_hbm,
          r_v, c_v, len_v, idx_v, val_v, acc_v, all_v, sem):
    cid = lax.axis_index("c")
    sid = lax.axis_index("s")
    wid = cid * NS + sid
    base = wid * PER

    pltpu.sync_copy(r_hbm.at[pl.ds(base, PER)], r_v)
    pltpu.sync_copy(c_hbm.at[pl.ds(base, PER)], c_v)
    pltpu.sync_copy(len_hbm.at[pl.ds(base, PER)], len_v)

    for j in range(NV):
        e0 = j * L
        jj, off = e0 // CH, e0 % CH
        rr = r_v[pl.ds(e0, L)]
        cc = c_v[pl.ds(e0, L)]
        idx_v[jj, pl.ds(off, L)] = rr * IMG + cc

    copies = [
        pltpu.async_copy(img_hbm.at[idx_v.at[jj]], val_v.at[jj], sem)
        for jj in range(NCH)
    ]
    for cp in copies:
        cp.wait()

    acc = jnp.zeros((L,), jnp.float32)
    for j in range(NV):
        e0 = j * L
        jj, off = e0 // CH, e0 % CH
        acc = acc + val_v[jj, pl.ds(off, L)] * len_v[pl.ds(e0, L)]
    acc_v[0, :] = acc

    pltpu.sync_copy(acc_v, parts_hbm.at[pl.ds(wid, 1)])
    plsc.subcore_barrier()

    @pl.when(sid == 0)
    def _():
        pltpu.sync_copy(parts_hbm.at[pl.ds(cid * NS, NS)], all_v)
        tot = jnp.zeros((L,), jnp.float32)
        for i in range(NS):
            tot = tot + all_v[i, :]
        lane = lax.iota(jnp.int32, L)
        for sh in (1, 2, 4, 8):
            tot = tot + tot.at[lane ^ sh].get(mode="promise_in_bounds")
        acc_v[0, :] = tot
        pltpu.sync_copy(acc_v, out_hbm.at[pl.ds(cid, 1)])


def kernel(img, lengths, idx):
    idx = idx.astype(jnp.int32)
    rows = idx[:, 0]
    cols = idx[:, 1]
    img_flat = img.reshape(-1)
    mesh = plsc.VectorSubcoreMesh(core_axis_name="c", subcore_axis_name="s")
    out, _ = pl.kernel(
        _body,
        mesh=mesh,
        out_type=(
            jax.ShapeDtypeStruct((NC, L), jnp.float32),
            jax.ShapeDtypeStruct((NW, L), jnp.float32),
        ),
        scratch_types=[
            pltpu.VMEM((PER,), jnp.int32),
            pltpu.VMEM((PER,), jnp.int32),
            pltpu.VMEM((PER,), jnp.float32),
            pltpu.VMEM((NCH, CH), jnp.int32),
            pltpu.VMEM((NCH, CH), jnp.float32),
            pltpu.VMEM((1, L), jnp.float32),
            pltpu.VMEM((NS, L), jnp.float32),
            pltpu.SemaphoreType.DMA,
        ],
    )(img_flat, rows, cols, lengths)
    return out[0, 0] + out[1, 0]


# speedup vs baseline: 7.8530x; 7.8530x over previous
import jax
import jax.numpy as jnp
from jax import lax
from jax.experimental import pallas as pl
from jax.experimental.pallas import tpu as pltpu
from jax.experimental.pallas import tpu_sc as plsc

IMG = 8192
N = 24576
NC = 2
NS = 16
L = 16
NW = NC * NS
PER = N // NW
CH = 128
NCH = PER // CH
NV = PER // L


def _body(img_hbm, r_hbm, c_hbm, len_hbm, out_hbm, parts_hbm,
          r_v, c_v, len_v, idx_v, val_v, acc_v, all_v, sem):
    cid = lax.axis_index("c")
    sid = lax.axis_index("s")
    wid = cid * NS + sid
    base = wid * PER

    pltpu.sync_copy(r_hbm.at[pl.ds(base, PER)], r_v)
    pltpu.sync_copy(c_hbm.at[pl.ds(base, PER)], c_v)
    pltpu.sync_copy(len_hbm.at[pl.ds(base, PER)], len_v)

    for j in range(NV):
        e0 = j * L
        jj, off = e0 // CH, e0 % CH
        rr = r_v[pl.ds(e0, L)]
        cc = c_v[pl.ds(e0, L)]
        addr = ((rr >> 3) << 16) + ((cc >> 7) << 10) + ((rr & 7) << 7) + (cc & 127)
        idx_v[jj, pl.ds(off, L)] = addr

    copies = [
        pltpu.async_copy(img_hbm.at[idx_v.at[jj]], val_v.at[jj], sem)
        for jj in range(NCH)
    ]
    for cp in copies:
        cp.wait()

    acc = jnp.zeros((L,), jnp.float32)
    for j in range(NV):
        e0 = j * L
        jj, off = e0 // CH, e0 % CH
        acc = acc + val_v[jj, pl.ds(off, L)] * len_v[pl.ds(e0, L)]
    acc_v[0, :] = acc

    pltpu.sync_copy(acc_v, parts_hbm.at[pl.ds(wid, 1)])
    plsc.subcore_barrier()

    @pl.when(sid == 0)
    def _():
        pltpu.sync_copy(parts_hbm.at[pl.ds(cid * NS, NS)], all_v)
        tot = jnp.zeros((L,), jnp.float32)
        for i in range(NS):
            tot = tot + all_v[i, :]
        lane = lax.iota(jnp.int32, L)
        for sh in (1, 2, 4, 8):
            tot = tot + tot.at[lane ^ sh].get(mode="promise_in_bounds")
        acc_v[0, :] = tot
        pltpu.sync_copy(acc_v, out_hbm.at[pl.ds(cid, 1)])


def kernel(img, lengths, idx):
    idx = idx.astype(jnp.int32)
    rows = idx[:, 0]
    cols = idx[:, 1]
    img_flat = (
        img.reshape(IMG // 8, 8, IMG // 128, 128)
        .transpose(0, 2, 1, 3)
        .reshape(-1)
    )
    mesh = plsc.VectorSubcoreMesh(core_axis_name="c", subcore_axis_name="s")
    out, _ = pl.kernel(
        _body,
        mesh=mesh,
        out_type=(
            jax.ShapeDtypeStruct((NC, L), jnp.float32),
            jax.ShapeDtypeStruct((NW, L), jnp.float32),
        ),
        scratch_types=[
            pltpu.VMEM((PER,), jnp.int32),
            pltpu.VMEM((PER,), jnp.int32),
            pltpu.VMEM((PER,), jnp.float32),
            pltpu.VMEM((NCH, CH), jnp.int32),
            pltpu.VMEM((NCH, CH), jnp.float32),
            pltpu.VMEM((1, L), jnp.float32),
            pltpu.VMEM((NS, L), jnp.float32),
            pltpu.SemaphoreType.DMA,
        ],
    )(img_flat, rows, cols, lengths)
    return out[0, 0] + out[1, 0]


# speedup vs baseline: 8.2637x; 1.0523x over previous
import jax
import jax.numpy as jnp
from jax import lax
from jax.experimental import pallas as pl
from jax.experimental.pallas import tpu as pltpu
from jax.experimental.pallas import tpu_sc as plsc

IMG = 8192
N = 24576
NC = 2
NS = 16
L = 16
NW = NC * NS
PER = N // NW
CH = 128
NCH = PER // CH
NV = PER // L


def _body(img_hbm, r_hbm, c_hbm, len_hbm, out_hbm, parts_hbm,
          r_v, c_v, len_v, idx_v, val_v, acc_v, all_v,
          sem_r, sem_c, sem_len, *gsems):
    cid = lax.axis_index("c")
    sid = lax.axis_index("s")
    wid = cid * NS + sid
    base = wid * PER

    cp_r = pltpu.async_copy(r_hbm.at[pl.ds(base, PER)], r_v, sem_r)
    cp_c = pltpu.async_copy(c_hbm.at[pl.ds(base, PER)], c_v, sem_c)
    cp_len = pltpu.async_copy(len_hbm.at[pl.ds(base, PER)], len_v, sem_len)
    cp_r.wait()
    cp_c.wait()

    gathers = []
    for jj in range(NCH):
        for k in range(CH // L):
            e0 = jj * CH + k * L
            rr = r_v[pl.ds(e0, L)]
            cc = c_v[pl.ds(e0, L)]
            addr = ((rr >> 3) << 16) + ((cc >> 7) << 10) + ((rr & 7) << 7) + (cc & 127)
            idx_v[jj, pl.ds(k * L, L)] = addr
        gathers.append(
            pltpu.async_copy(img_hbm.at[idx_v.at[jj]], val_v.at[jj], gsems[jj]))

    cp_len.wait()

    acc = jnp.zeros((L,), jnp.float32)
    for jj in range(NCH):
        gathers[jj].wait()
        for k in range(CH // L):
            e0 = jj * CH + k * L
            acc = acc + val_v[jj, pl.ds(k * L, L)] * len_v[pl.ds(e0, L)]
    acc_v[0, :] = acc

    pltpu.sync_copy(acc_v, parts_hbm.at[pl.ds(wid, 1)])
    plsc.subcore_barrier()

    @pl.when(sid == 0)
    def _():
        pltpu.sync_copy(parts_hbm.at[pl.ds(cid * NS, NS)], all_v)
        tot = jnp.zeros((L,), jnp.float32)
        for i in range(NS):
            tot = tot + all_v[i, :]
        lane = lax.iota(jnp.int32, L)
        for sh in (1, 2, 4, 8):
            tot = tot + tot.at[lane ^ sh].get(mode="promise_in_bounds")
        acc_v[0, :] = tot
        pltpu.sync_copy(acc_v, out_hbm.at[pl.ds(cid, 1)])


def kernel(img, lengths, idx):
    idx = idx.astype(jnp.int32)
    rows = idx[:, 0]
    cols = idx[:, 1]
    img_flat = (
        img.reshape(IMG // 8, 8, IMG // 128, 128)
        .transpose(0, 2, 1, 3)
        .reshape(-1)
    )
    mesh = plsc.VectorSubcoreMesh(core_axis_name="c", subcore_axis_name="s")
    out, _ = pl.kernel(
        _body,
        mesh=mesh,
        out_type=(
            jax.ShapeDtypeStruct((NC, L), jnp.float32),
            jax.ShapeDtypeStruct((NW, L), jnp.float32),
        ),
        scratch_types=[
            pltpu.VMEM((PER,), jnp.int32),
            pltpu.VMEM((PER,), jnp.int32),
            pltpu.VMEM((PER,), jnp.float32),
            pltpu.VMEM((NCH, CH), jnp.int32),
            pltpu.VMEM((NCH, CH), jnp.float32),
            pltpu.VMEM((1, L), jnp.float32),
            pltpu.VMEM((NS, L), jnp.float32),
        ] + [pltpu.SemaphoreType.DMA] * (3 + NCH),
    )(img_flat, rows, cols, lengths)
    return out[0, 0] + out[1, 0]
